# baseline (speedup 1.0000x reference)
import functools

import numpy as np

import jax
import jax.numpy as jnp
from jax import lax
from jax.experimental import pallas as pl
from jax.experimental.pallas import tpu as pltpu
from jax.experimental.pallas import tpu_sc as plsc

N = 10000
E = 320000
D = 128
NOUT = 128

NC = 2
NS = 16
NW = NC * NS
EPT = E // NW
K = 80
NCHUNK = EPT // K
RPT = N // NS

_mesh = plsc.VectorSubcoreMesh(
    core_axis_name="c", subcore_axis_name="s", num_cores=NC, num_subcores=NS
)


def _gather_start(table, sb, buf, sem):
    pltpu.async_copy(table.at[sb], buf, sem)


def _gather_wait(table, sb, buf, sem):
    pltpu.make_async_copy(table.at[sb], buf, sem).wait()


def _scale(buf, dwb):
    def scale(e, carry):
        wv = plsc.bitcast(dwb[1, pl.ds(e, 16)], jnp.float32)
        wvec = jnp.full((16,), wv[0], jnp.float32)
        for cix in range(8):
            sl = pl.ds(cix * 16, 16)
            buf[e, sl] = buf[e, sl] * wvec
        return carry

    lax.fori_loop(0, K, scale, 0, unroll=4)


def _scat_start(buf, dwb, acc, sem):
    pltpu.async_copy(buf, acc.at[dwb.at[0]], sem, add=True)


def _scat_wait(buf, dwb, acc, sem):
    pltpu.make_async_copy(buf, acc.at[dwb.at[0]], sem).wait()


def _src_start(spk, wid, ci, sb, sem):
    pltpu.async_copy(spk.at[wid, ci], sb, sem)


def _src_wait(spk, wid, sb, sem):
    pltpu.make_async_copy(spk.at[wid, 0], sb, sem).wait()


def _dw_start(dpk, wid, ci, dwb, sem):
    pltpu.async_copy(dpk.at[wid, ci], dwb.at[pl.ds(0, 2)], sem)


def _dw_wait(dpk, wid, dwb, sem):
    pltpu.make_async_copy(dpk.at[wid, 0], dwb.at[pl.ds(0, 2)], sem).wait()


def _spmm_body(table, spk, dpk, out, acc, rb, sbuf, dwb, gs, ss, srs, dws):
    c = lax.axis_index("c")
    s = lax.axis_index("s")
    wid = c * NS + s

    for q in range(4):
        _src_start(spk, wid, q, sbuf[q], srs[q])
    _dw_start(dpk, wid, 0, dwb[0], dws[0])
    _dw_start(dpk, wid, 1, dwb[1], dws[1])

    zero = jnp.zeros((16,), jnp.float32)

    def zrow(r, carry):
        for cix in range(8):
            rb[0][r, pl.ds(cix * 16, 16)] = zero
        return carry

    lax.fori_loop(0, K, zrow, 0)
    base_r = s * RPT
    for j in range(7):
        pltpu.sync_copy(rb[0], acc.at[pl.ds(base_r + j * K, K)])
    pltpu.sync_copy(rb[0].at[pl.ds(0, 65)], acc.at[pl.ds(base_r + 560, 65)])

    _src_wait(spk, wid, sbuf[0], srs[0])
    _gather_start(table, sbuf[0], rb[0], gs[0])
    _src_wait(spk, wid, sbuf[1], srs[1])
    _gather_start(table, sbuf[1], rb[1], gs[1])
    plsc.subcore_barrier()

    def proc(i, q, first_pair):
        q2 = (q + 2) % 4
        _gather_wait(table, sbuf[q], rb[q], gs[q])
        if i + 4 <= NCHUNK:
            _src_start(spk, wid, i + 4, sbuf[q], srs[q])
        if not first_pair:
            _scat_wait(rb[q2], dwb[q2], acc, ss[q2])
        if i + 2 < NCHUNK:
            _dw_start(dpk, wid, i + 2, dwb[q2], dws[q2])
            _src_wait(spk, wid, sbuf[q2], srs[q2])
            _gather_start(table, sbuf[q2], rb[q2], gs[q2])
        _dw_wait(dpk, wid, dwb[q], dws[q])
        _scale(rb[q], dwb[q])
        _scat_start(rb[q], dwb[q], acc, ss[q])

    def proc_dyn(i, q):
        q2 = (q + 2) % 4
        _gather_wait(table, sbuf[q], rb[q], gs[q])
        _src_start(spk, wid, i + 4, sbuf[q], srs[q])
        _scat_wait(rb[q2], dwb[q2], acc, ss[q2])
        _dw_start(dpk, wid, i + 2, dwb[q2], dws[q2])
        _src_wait(spk, wid, sbuf[q2], srs[q2])
        _gather_start(table, sbuf[q2], rb[q2], gs[q2])
        _dw_wait(dpk, wid, dwb[q], dws[q])
        _scale(rb[q], dwb[q])
        _scat_start(rb[q], dwb[q], acc, ss[q])

    proc(0, 0, True)
    proc(1, 1, True)

    def body(j, carry):
        i0 = 4 * j + 2
        for q0 in range(4):
            proc_dyn(i0 + q0, (2 + q0) % 4)
        return carry

    lax.fori_loop(0, 30, body, 0)
    proc(122, 2, False)
    proc(123, 3, False)
    proc(124, 0, False)
    _scat_wait(rb[3], dwb[3], acc, ss[3])
    _scat_wait(rb[0], dwb[0], acc, ss[0])
    _src_wait(spk, wid, sbuf[1], srs[1])
    plsc.subcore_barrier()

    start = pl.multiple_of(s * RPT - lax.rem(s, 8), 8)
    pltpu.sync_copy(
        acc.at[pl.ds(start, RPT + 7)],
        out.at[pl.ds(pl.multiple_of(c * N + start, 8), RPT + 7)],
    )


_spmm_sc = pl.kernel(
    _spmm_body,
    out_type=jax.ShapeDtypeStruct((NC * N, D), jnp.float32),
    mesh=_mesh,
    scratch_types=[
        pltpu.VMEM_SHARED((N, D), jnp.float32),
        [pltpu.VMEM((K, D), jnp.float32)] * 4,
        [pltpu.VMEM((K,), jnp.int32)] * 4,
        [pltpu.VMEM((3, K), jnp.int32)] * 4,
        [pltpu.SemaphoreType.DMA] * 4,
        [pltpu.SemaphoreType.DMA] * 4,
        [pltpu.SemaphoreType.DMA] * 4,
        [pltpu.SemaphoreType.DMA] * 4,
    ],
    compiler_params=pltpu.CompilerParams(
        use_tc_tiling_on_sc=False, needs_layout_passes=False
    ),
)


def _elu(t):
    return jnp.where(t > 0, t, jnp.exp(jnp.minimum(t, 0.0)) - 1.0)


def _combine_body(p0, p1, o):
    o[...] = _elu(p0[...] + p1[...])


def _final_body(p0, p1, wt, bias, o):
    h = _elu(p0[...] + p1[...])
    o[...] = (
        lax.dot_general(
            h, wt[...], (((1,), (1,)), ((), ())),
            preferred_element_type=jnp.float32,
        )
        + bias[...]
    )


BR = 1000


def _combine(partials):
    return pl.pallas_call(
        _combine_body,
        grid=(N // BR,),
        in_specs=[
            pl.BlockSpec((BR, D), lambda i: (i, 0)),
            pl.BlockSpec((BR, D), lambda i: (i + N // BR, 0)),
        ],
        out_specs=pl.BlockSpec((BR, D), lambda i: (i, 0)),
        out_shape=jax.ShapeDtypeStruct((N, D), jnp.float32),
    )(partials, partials)


def _final(partials, W, b2):
    return pl.pallas_call(
        _final_body,
        grid=(N // BR,),
        in_specs=[
            pl.BlockSpec((BR, D), lambda i: (i, 0)),
            pl.BlockSpec((BR, D), lambda i: (i + N // BR, 0)),
            pl.BlockSpec((NOUT, D), lambda i: (0, 0)),
            pl.BlockSpec((1, NOUT), lambda i: (0, 0)),
        ],
        out_specs=pl.BlockSpec((BR, NOUT), lambda i: (i, 0)),
        out_shape=jax.ShapeDtypeStruct((N, NOUT), jnp.float32),
    )(partials, partials, W, b2)


@jax.jit
def kernel(x, edge_index, edge_weight, scalar0, scalar1, W, b):
    dst = edge_index[0]
    src = edge_index[1]
    src3 = src.reshape(NW, NCHUNK, K)
    dst3 = dst.reshape(NW, NCHUNK, K)
    spk = jnp.concatenate(
        [src3, jnp.zeros((NW, 1, K), jnp.int32)], axis=1
    )
    w1 = lax.bitcast_convert_type(
        (edge_weight * scalar0[0]).reshape(NW, NCHUNK, K), jnp.int32
    )
    w2 = lax.bitcast_convert_type(
        (edge_weight * scalar1[0]).reshape(NW, NCHUNK, K), jnp.int32
    )
    dpk1 = jnp.stack([dst3, w1], axis=2)
    dpk2 = jnp.stack([dst3, w2], axis=2)
    p1 = _spmm_sc(x, spk, dpk1)
    h1 = _combine(p1)
    p2 = _spmm_sc(h1, spk, dpk2)
    return _final(p2, W, b.reshape(1, NOUT))


# speedup vs baseline: 1.1655x; 1.1655x over previous
import functools

import numpy as np

import jax
import jax.numpy as jnp
from jax import lax
from jax.experimental import pallas as pl
from jax.experimental.pallas import tpu as pltpu
from jax.experimental.pallas import tpu_sc as plsc

N = 10000
E = 320000
D = 128
NOUT = 128

NC = 2
NS = 16
NW = NC * NS
EPT = E // NW
K = 80
NCHUNK = EPT // K
RPT = N // NS

_mesh = plsc.VectorSubcoreMesh(
    core_axis_name="c", subcore_axis_name="s", num_cores=NC, num_subcores=NS
)


def _gather_start(table, sb, buf, sem):
    pltpu.async_copy(table.at[sb], buf, sem)


def _gather_wait(table, sb, buf, sem):
    pltpu.make_async_copy(table.at[sb], buf, sem).wait()


def _scale(buf, wb):
    def scale(e, carry):
        wvec = jnp.full((16,), wb[pl.ds(e, 16)][0], jnp.float32)
        for cix in range(8):
            sl = pl.ds(cix * 16, 16)
            buf[e, sl] = buf[e, sl] * wvec
        return carry

    lax.fori_loop(0, K, scale, 0, unroll=4)


def _scat_start(buf, dstb, acc, sem):
    pltpu.async_copy(buf, acc.at[dstb], sem, add=True)


def _scat_wait(buf, dstb, acc, sem):
    pltpu.make_async_copy(buf, acc.at[dstb], sem).wait()


def _src_start(spk, wid, ci, sb, sem):
    pltpu.async_copy(spk.at[wid, ci], sb, sem)


def _src_wait(spk, wid, sb, sem):
    pltpu.make_async_copy(spk.at[wid, 0], sb, sem).wait()


def _w_start(wpk, wid, ci, wb, sem):
    pltpu.async_copy(wpk.at[wid, ci], wb.at[pl.ds(0, K)], sem)


def _w_wait(wpk, wid, wb, sem):
    pltpu.make_async_copy(wpk.at[wid, 0], wb.at[pl.ds(0, K)], sem).wait()


def _spmm_body(table, spk, dpk, wpk, out, acc, rb, sbuf, dstb, wb, gs, ss,
               srs, sds, sws):
    c = lax.axis_index("c")
    s = lax.axis_index("s")
    wid = c * NS + s

    for q in range(4):
        _src_start(spk, wid, q, sbuf[q], srs[q])
    for q in range(2):
        _src_start(dpk, wid, q, dstb[q], sds[q])
        _w_start(wpk, wid, q, wb[q], sws[q])
    del q

    zero = jnp.zeros((16,), jnp.float32)

    def zrow(r, carry):
        for cix in range(8):
            rb[0][r, pl.ds(cix * 16, 16)] = zero
        return carry

    lax.fori_loop(0, K, zrow, 0)
    base_r = s * RPT
    for j in range(7):
        pltpu.sync_copy(rb[0], acc.at[pl.ds(base_r + j * K, K)])
    pltpu.sync_copy(rb[0].at[pl.ds(0, 65)], acc.at[pl.ds(base_r + 560, 65)])

    _src_wait(spk, wid, sbuf[0], srs[0])
    _gather_start(table, sbuf[0], rb[0], gs[0])
    _src_wait(spk, wid, sbuf[1], srs[1])
    _gather_start(table, sbuf[1], rb[1], gs[1])
    plsc.subcore_barrier()

    def proc(i, q, first_pair):
        q2 = (q + 2) % 4
        _gather_wait(table, sbuf[q], rb[q], gs[q])
        if i + 4 < NCHUNK:
            _src_start(spk, wid, i + 4, sbuf[q], srs[q])
        if not first_pair:
            _scat_wait(rb[q2], dstb[q2], acc, ss[q2])
        if i + 2 < NCHUNK:
            _src_start(dpk, wid, i + 2, dstb[q2], sds[q2])
            _w_start(wpk, wid, i + 2, wb[q2], sws[q2])
            _src_wait(spk, wid, sbuf[q2], srs[q2])
            _gather_start(table, sbuf[q2], rb[q2], gs[q2])
        _src_wait(dpk, wid, dstb[q], sds[q])
        _w_wait(wpk, wid, wb[q], sws[q])
        _scale(rb[q], wb[q])
        _scat_start(rb[q], dstb[q], acc, ss[q])

    def proc_dyn(i, q):
        q2 = (q + 2) % 4
        _gather_wait(table, sbuf[q], rb[q], gs[q])
        _src_start(spk, wid, jnp.minimum(i + 4, NCHUNK - 1), sbuf[q], srs[q])
        _scat_wait(rb[q2], dstb[q2], acc, ss[q2])
        _src_start(dpk, wid, i + 2, dstb[q2], sds[q2])
        _w_start(wpk, wid, i + 2, wb[q2], sws[q2])
        _src_wait(spk, wid, sbuf[q2], srs[q2])
        _gather_start(table, sbuf[q2], rb[q2], gs[q2])
        _src_wait(dpk, wid, dstb[q], sds[q])
        _w_wait(wpk, wid, wb[q], sws[q])
        _scale(rb[q], wb[q])
        _scat_start(rb[q], dstb[q], acc, ss[q])

    proc(0, 0, True)
    proc(1, 1, True)

    def body(j, carry):
        i0 = 4 * j + 2
        for q0 in range(4):
            proc_dyn(i0 + q0, (2 + q0) % 4)
        return carry

    lax.fori_loop(0, 30, body, 0)
    proc(122, 2, False)
    proc(123, 3, False)
    proc(124, 0, False)
    _scat_wait(rb[3], dstb[1], acc, ss[3])
    _scat_wait(rb[0], dstb[0], acc, ss[0])
    _src_wait(spk, wid, sbuf[1], srs[1])
    plsc.subcore_barrier()

    start = pl.multiple_of(s * RPT - lax.rem(s, 8), 8)
    pltpu.sync_copy(
        acc.at[pl.ds(start, RPT + 7)],
        out.at[pl.ds(pl.multiple_of(c * N + start, 8), RPT + 7)],
    )


_spmm_sc = pl.kernel(
    _spmm_body,
    out_type=jax.ShapeDtypeStruct((NC * N, D), jnp.float32),
    mesh=_mesh,
    scratch_types=[
        pltpu.VMEM_SHARED((N, D), jnp.float32),
        [pltpu.VMEM((K, D), jnp.float32)] * 4,
        [pltpu.VMEM((K,), jnp.int32)] * 4,
        [pltpu.VMEM((K,), jnp.int32)] * 4,
        [pltpu.VMEM((K + 16,), jnp.float32)] * 4,
        [pltpu.SemaphoreType.DMA] * 4,
        [pltpu.SemaphoreType.DMA] * 4,
        [pltpu.SemaphoreType.DMA] * 4,
        [pltpu.SemaphoreType.DMA] * 4,
        [pltpu.SemaphoreType.DMA] * 4,
    ],
    compiler_params=pltpu.CompilerParams(
        use_tc_tiling_on_sc=False, needs_layout_passes=False
    ),
)


def _elu(t):
    return jnp.where(t > 0, t, jnp.exp(jnp.minimum(t, 0.0)) - 1.0)


def _combine_body(p0, p1, o):
    o[...] = _elu(p0[...] + p1[...])


def _final_body(p0, p1, wt, bias, o):
    h = _elu(p0[...] + p1[...])
    o[...] = (
        lax.dot_general(
            h, wt[...], (((1,), (1,)), ((), ())),
            preferred_element_type=jnp.float32,
        )
        + bias[...]
    )


BR = 1000


def _combine(partials):
    return pl.pallas_call(
        _combine_body,
        grid=(N // BR,),
        in_specs=[
            pl.BlockSpec((BR, D), lambda i: (i, 0)),
            pl.BlockSpec((BR, D), lambda i: (i + N // BR, 0)),
        ],
        out_specs=pl.BlockSpec((BR, D), lambda i: (i, 0)),
        out_shape=jax.ShapeDtypeStruct((N, D), jnp.float32),
    )(partials, partials)


def _final(partials, W, b2):
    return pl.pallas_call(
        _final_body,
        grid=(N // BR,),
        in_specs=[
            pl.BlockSpec((BR, D), lambda i: (i, 0)),
            pl.BlockSpec((BR, D), lambda i: (i + N // BR, 0)),
            pl.BlockSpec((NOUT, D), lambda i: (0, 0)),
            pl.BlockSpec((1, NOUT), lambda i: (0, 0)),
        ],
        out_specs=pl.BlockSpec((BR, NOUT), lambda i: (i, 0)),
        out_shape=jax.ShapeDtypeStruct((N, NOUT), jnp.float32),
    )(partials, partials, W, b2)


@jax.jit
def kernel(x, edge_index, edge_weight, scalar0, scalar1, W, b):
    dst = edge_index[0]
    src = edge_index[1]
    spk = src.reshape(NW, NCHUNK, K)
    dpk = dst.reshape(NW, NCHUNK, K)
    wpk1 = (edge_weight * scalar0[0]).reshape(NW, NCHUNK, K)
    wpk2 = (edge_weight * scalar1[0]).reshape(NW, NCHUNK, K)
    p1 = _spmm_sc(x, spk, dpk, wpk1)
    h1 = _combine(p1)
    p2 = _spmm_sc(h1, spk, dpk, wpk2)
    return _final(p2, W, b.reshape(1, NOUT))
